# Initial kernel scaffold; baseline (speedup 1.0000x reference)
import jax
import jax.numpy as jnp
from jax.experimental import pallas as pl


def kernel(im, im_large, W1, W2):
    raise NotImplementedError("write your pallas kernel here")


# speedup vs baseline: 9.9660x; 9.9660x over previous
import math

import jax
import jax.numpy as jnp
import numpy as np
from jax.experimental import pallas as pl

_NCLS = 11
_SZ = 512
_BLK = 128

_yy, _xx = np.meshgrid(np.arange(_SZ), np.arange(_SZ), indexing='ij')
_dd = np.sqrt((_xx - _SZ / 2 - 0.001) ** 2 + (_yy - _SZ / 2 - 0.001) ** 2)
_DISK = jnp.asarray(((_dd > 0) & (_dd <= 100.0)).astype(np.float32))


def _msum_body(im_ref, w1_ref, o_ref):
    x = im_ref[0].reshape(3, -1)
    w1 = w1_ref[...]
    fm = jnp.dot(w1.T, x, preferred_element_type=jnp.float32)
    o_ref[0] = jnp.sum(fm[1:], axis=0).reshape(_BLK, _SZ)


def _comp_body(mask_ref, im_ref, hd_ref, w1_ref, w2_ref, o_ref):
    x = im_ref[0].reshape(3, -1)
    h = hd_ref[0].reshape(3, -1)
    w1 = w1_ref[...]
    w2 = w2_ref[...]
    fm = jnp.dot(w1.T, x, preferred_element_type=jnp.float32)
    hm = jnp.dot(w2.T, h, preferred_element_type=jnp.float32)
    m = mask_ref[...].reshape(1, -1)
    sel = jnp.where(m > 0.5, hm, fm)
    mx = jnp.max(sel, axis=0, keepdims=True)
    e = jnp.exp(sel - mx)
    sm = e / jnp.sum(e, axis=0, keepdims=True)
    o_ref[0] = sm.reshape(_NCLS, _BLK, _SZ)


def _retina_window(index, image):
    w = _SZ
    ix = (index % w).astype(jnp.int32)
    iy = (index // w).astype(jnp.int32)
    fx_l = jnp.clip(ix * 16, 0, 2047).astype(jnp.float32)
    fy_l = jnp.clip(iy * 16, 0, 2047).astype(jnp.float32)
    fx_r = jnp.clip(ix * 4, 0, 511)
    fy_r = jnp.clip(iy * 4, 0, 511)
    oy = jnp.clip(fy_r - 128, 0, 256)
    ox = jnp.clip(fx_r - 128, 0, 256)
    wy, wx = jnp.meshgrid(jnp.arange(256, dtype=jnp.float32),
                          jnp.arange(256, dtype=jnp.float32), indexing='ij')
    edge3 = math.sqrt(2.0) * 256.0

    def per_batch(img, oy_i, ox_i, fx_i, fy_i):
        py = oy_i.astype(jnp.float32) + wy
        px = ox_i.astype(jnp.float32) + wx
        dx = px - 256.001
        dy = py - 256.001
        dist = jnp.sqrt(dx ** 2 + dy ** 2)

        def g(sx, sy):
            sxi = jnp.clip(jnp.round(sx), 0, 2047).astype(jnp.int32)
            syi = jnp.clip(jnp.round(sy), 0, 2047).astype(jnp.int32)
            return img[:, syi, sxi]

        def gpool(sx, sy, k):
            sxi = jnp.clip(jnp.round(sx), 0, 2048 - k).astype(jnp.int32)
            syi = jnp.clip(jnp.round(sy), 0, 2048 - k).astype(jnp.int32)
            acc = jnp.zeros((3, 256, 256), img.dtype)
            for i in range(k):
                for j in range(k):
                    acc = acc + img[:, syi + i, sxi + j]
            return acc / float(k * k)

        v0 = g(fx_i + dx, fy_i + dy)

        def rel_of(f, edge):
            return (f * dist - edge) / jnp.maximum(dist, 1e-6)

        r1 = rel_of(2.0, 180.0)
        v1 = gpool(dx * r1 + fx_i, dy * r1 + fy_i, 2)
        r2 = rel_of(3.0, 250.0)
        v2 = gpool(dx * r2 + fx_i, dy * r2 + fy_i, 3)
        r3 = rel_of(4.0, edge3)
        v3 = gpool(dx * r3 + fx_i, dy * r3 + fy_i, 4)

        m0 = (dist > 0) & (dist <= 100.0)
        m1 = (dist > 100.0) & (dist <= 180.0)
        m2 = (dist > 180.0) & (dist <= 250.0)
        m3 = (dist > 250.0) & (dist <= edge3)
        out = jnp.where(m0[None], v0, 0.0)
        out = jnp.where(m1[None], v1, out)
        out = jnp.where(m2[None], v2, out)
        out = jnp.where(m3[None], v3, out)
        return out

    hd_win = jax.vmap(per_batch)(image, oy, ox, fx_l, fy_l)
    sy0 = fy_r - 256 - oy
    sx0 = fx_r - 256 - ox
    return hd_win, sy0, sx0


@jax.jit
def _fwd(im, im_large, W1, W2):
    b = im.shape[0]
    nrow = _SZ // _BLK

    msum = pl.pallas_call(
        _msum_body,
        grid=(b, nrow),
        in_specs=[
            pl.BlockSpec((1, 3, _BLK, _SZ), lambda i, j: (i, 0, j, 0)),
            pl.BlockSpec((3, _NCLS), lambda i, j: (0, 0)),
        ],
        out_specs=pl.BlockSpec((1, _BLK, _SZ), lambda i, j: (i, j, 0)),
        out_shape=jax.ShapeDtypeStruct((b, _SZ, _SZ), jnp.float32),
    )(im, W1)

    index = jnp.argmax(msum.reshape(b, -1), axis=1)

    hd_win, sy0, sx0 = _retina_window(index, im_large)

    pad = jnp.pad(hd_win, ((0, 0), (0, 0), (256, 255), (256, 255)), mode='edge')

    def sl(p, y0, x0):
        return jax.lax.dynamic_slice(p, (0, 256 + y0, 256 + x0), (3, _SZ, _SZ))

    hd_shift = jax.vmap(sl)(pad, sy0, sx0)

    out = pl.pallas_call(
        _comp_body,
        grid=(b, nrow),
        in_specs=[
            pl.BlockSpec((_BLK, _SZ), lambda i, j: (j, 0)),
            pl.BlockSpec((1, 3, _BLK, _SZ), lambda i, j: (i, 0, j, 0)),
            pl.BlockSpec((1, 3, _BLK, _SZ), lambda i, j: (i, 0, j, 0)),
            pl.BlockSpec((3, _NCLS), lambda i, j: (0, 0)),
            pl.BlockSpec((3, _NCLS), lambda i, j: (0, 0)),
        ],
        out_specs=pl.BlockSpec((1, _NCLS, _BLK, _SZ), lambda i, j: (i, 0, j, 0)),
        out_shape=jax.ShapeDtypeStruct((b, _NCLS, _SZ, _SZ), jnp.float32),
    )(_DISK, im, hd_shift, W1, W2)
    return out


def kernel(im, im_large, W1, W2):
    return _fwd(im, im_large, W1, W2)
